# baseline (speedup 1.0000x reference)
import functools

import jax
import jax.numpy as jnp
from jax import lax
from jax.experimental import pallas as pl
from jax.experimental.pallas import tpu as pltpu
from jax.experimental.pallas import tpu_sc as plsc

B, T, D, K = 8, 576, 256, 8192
N = B * T
TB = 512
NB = N // TB
KT = 4096
NKT = K // KT

NC = 2
NS = 16
NW = NC * NS
BPW = N // NW
NCH = 2
CH = BPW // NCH


def _fold_min(x):
    n = x.shape[1]
    while n > 128:
        h = n // 2
        x = jnp.minimum(x[:, :h], x[:, h:])
        n = h
    return jnp.min(x, axis=1)


def _argmin_into(r, cb_ref, c2_ref, idx_ref):
    @pl.when(pl.program_id(0) == 0)
    def _():
        for k in range(NKT):
            cbt = cb_ref[pl.ds(k * KT, KT), :]
            c2_ref[0, pl.ds(k * KT, KT)] = jnp.sum(cbt * cbt, axis=1)

    r2 = jnp.sum(r * r, axis=1)
    rd = r + r
    minv = jnp.full((TB,), jnp.inf, dtype=jnp.float32)
    mini = jnp.zeros((TB,), dtype=jnp.int32)
    io = lax.broadcasted_iota(jnp.int32, (TB, KT), 1)
    for k in range(NKT):
        cbt = cb_ref[pl.ds(k * KT, KT), :]
        c2 = c2_ref[0, pl.ds(k * KT, KT)]
        cross2 = lax.dot_general(rd, cbt, (((1,), (1,)), ((), ())),
                                 preferred_element_type=jnp.float32)
        d2 = (r2[:, None] + c2[None, :]) - cross2
        tmin = _fold_min(d2)
        targ = _fold_min(jnp.where(d2 == tmin[:, None], io, K)) + k * KT
        upd = tmin < minv
        minv = jnp.where(upd, tmin, minv)
        mini = jnp.where(upd, targ, mini)
    idx_ref[0, 0, :] = mini


def _vq_first_body(r_ref, cb_ref, idx_ref, c2_ref):
    _argmin_into(r_ref[...], cb_ref, c2_ref, idx_ref)


def _vq_next_body(r_ref, rows_ref, cb_ref, idx_ref, rout_ref, c2_ref):
    r = r_ref[...] - rows_ref[...]
    rout_ref[...] = r
    _argmin_into(r, cb_ref, c2_ref, idx_ref)


def _vq_last_body(r_ref, rows_ref, x_ref, cb_ref, idx_ref, rec01_ref, c2_ref):
    r = r_ref[...] - rows_ref[...]
    rec01_ref[...] = x_ref[...] - r
    _argmin_into(r, cb_ref, c2_ref, idx_ref)


def _vq_first(r, cb):
    return pl.pallas_call(
        _vq_first_body,
        grid=(NB,),
        in_specs=[
            pl.BlockSpec((TB, D), lambda i: (i, 0)),
            pl.BlockSpec((K, D), lambda i: (0, 0)),
        ],
        out_specs=pl.BlockSpec((1, 1, TB), lambda i: (i, 0, 0)),
        out_shape=jax.ShapeDtypeStruct((NB, 1, TB), jnp.int32),
        scratch_shapes=[pltpu.VMEM((1, K), jnp.float32)],
    )(r, cb)


def _vq_next(r_prev, rows_prev, cb):
    return pl.pallas_call(
        _vq_next_body,
        grid=(NB,),
        in_specs=[
            pl.BlockSpec((TB, D), lambda i: (i, 0)),
            pl.BlockSpec((TB, D), lambda i: (i, 0)),
            pl.BlockSpec((K, D), lambda i: (0, 0)),
        ],
        out_specs=[
            pl.BlockSpec((1, 1, TB), lambda i: (i, 0, 0)),
            pl.BlockSpec((TB, D), lambda i: (i, 0)),
        ],
        out_shape=[
            jax.ShapeDtypeStruct((NB, 1, TB), jnp.int32),
            jax.ShapeDtypeStruct((N, D), jnp.float32),
        ],
        scratch_shapes=[pltpu.VMEM((1, K), jnp.float32)],
    )(r_prev, rows_prev, cb)


def _vq_last(r_prev, rows_prev, x, cb):
    return pl.pallas_call(
        _vq_last_body,
        grid=(NB,),
        in_specs=[
            pl.BlockSpec((TB, D), lambda i: (i, 0)),
            pl.BlockSpec((TB, D), lambda i: (i, 0)),
            pl.BlockSpec((TB, D), lambda i: (i, 0)),
            pl.BlockSpec((K, D), lambda i: (0, 0)),
        ],
        out_specs=[
            pl.BlockSpec((1, 1, TB), lambda i: (i, 0, 0)),
            pl.BlockSpec((TB, D), lambda i: (i, 0)),
        ],
        out_shape=[
            jax.ShapeDtypeStruct((NB, 1, TB), jnp.int32),
            jax.ShapeDtypeStruct((N, D), jnp.float32),
        ],
        scratch_shapes=[pltpu.VMEM((1, K), jnp.float32)],
    )(r_prev, rows_prev, x, cb)


def _sc_gather_body(cb_hbm, idx_hbm, out_hbm, idx_v, rows_v, sem):
    wid = lax.axis_index("s") * NC + lax.axis_index("c")
    pltpu.sync_copy(idx_hbm.at[wid], idx_v)
    copies = [
        pltpu.async_copy(cb_hbm.at[idx_v.at[j]], rows_v.at[j], sem)
        for j in range(NCH)
    ]
    for c in copies:
        c.wait()
    pltpu.sync_copy(rows_v, out_hbm.at[wid])


def _sc_gather_add_body(cb_hbm, idx_hbm, base_hbm, out_hbm, idx_v, rows_v,
                        base_v, sem):
    wid = lax.axis_index("s") * NC + lax.axis_index("c")
    pltpu.sync_copy(idx_hbm.at[wid], idx_v)
    copies = [
        pltpu.async_copy(cb_hbm.at[idx_v.at[j]], rows_v.at[j], sem)
        for j in range(NCH)
    ]
    pltpu.sync_copy(base_hbm.at[wid], base_v)
    for c in copies:
        c.wait()

    def _add_row(i, _):
        for j in range(NCH):
            for l in range(D // 16):
                s = pl.ds(l * 16, 16)
                rows_v[j, i, s] = rows_v[j, i, s] + base_v[j, i, s]
        return 0

    lax.fori_loop(0, CH, _add_row, 0)
    pltpu.sync_copy(rows_v, out_hbm.at[wid])


@functools.cache
def _sc_gather_kernel():
    return pl.kernel(
        _sc_gather_body,
        out_type=jax.ShapeDtypeStruct((NW, NCH, CH, D), jnp.float32),
        mesh=plsc.VectorSubcoreMesh(core_axis_name="c", subcore_axis_name="s"),
        scratch_types=[
            pltpu.VMEM((NCH, CH), jnp.int32),
            pltpu.VMEM((NCH, CH, D), jnp.float32),
            pltpu.SemaphoreType.DMA,
        ],
    )


@functools.cache
def _sc_gather_add_kernel():
    return pl.kernel(
        _sc_gather_add_body,
        out_type=jax.ShapeDtypeStruct((NW, NCH, CH, D), jnp.float32),
        mesh=plsc.VectorSubcoreMesh(core_axis_name="c", subcore_axis_name="s"),
        scratch_types=[
            pltpu.VMEM((NCH, CH), jnp.int32),
            pltpu.VMEM((NCH, CH, D), jnp.float32),
            pltpu.VMEM((NCH, CH, D), jnp.float32),
            pltpu.SemaphoreType.DMA,
        ],
    )


def _gather(cb, idx_flat):
    out = _sc_gather_kernel()(cb, idx_flat.reshape(NW, NCH, CH))
    return out.reshape(N, D)


def _gather_add(cb, idx_flat, base):
    out = _sc_gather_add_kernel()(
        cb, idx_flat.reshape(NW, NCH, CH), base.reshape(NW, NCH, CH, D))
    return out.reshape(N, D)


def kernel(multimodal_features, codebook_0, codebook_1, codebook_2):
    x = multimodal_features.reshape(N, D)

    idx0 = _vq_first(x, codebook_0).reshape(N)
    rows0 = _gather(codebook_0, idx0)

    idx1, r1 = _vq_next(x, rows0, codebook_1)
    idx1 = idx1.reshape(N)
    rows1 = _gather(codebook_1, idx1)

    idx2, rec01 = _vq_last(r1, rows1, x, codebook_2)
    idx2 = idx2.reshape(N)
    recon = _gather_add(codebook_2, idx2, rec01).reshape(B, T, D)
    semantic_ids = jnp.stack([idx0, idx1, idx2], axis=-1).reshape(B, T, 3)
    return semantic_ids, recon


# speedup vs baseline: 1.0255x; 1.0255x over previous
import functools

import jax
import jax.numpy as jnp
from jax import lax
from jax.experimental import pallas as pl
from jax.experimental.pallas import tpu as pltpu
from jax.experimental.pallas import tpu_sc as plsc

B, T, D, K = 8, 576, 256, 8192
N = B * T
TB = 512
NB = N // TB
KT = 2048
NKT = K // KT

NC = 2
NS = 16
NW = NC * NS
BPW = N // NW
NCH = 2
CH = BPW // NCH


def _fold_min(x):
    n = x.shape[1]
    while n > 128:
        h = n // 2
        x = jnp.minimum(x[:, :h], x[:, h:])
        n = h
    return jnp.min(x, axis=1)


def _argmin_into(r, cb_ref, c2_ref, idx_ref):
    r2 = jnp.sum(r * r, axis=1)
    rd = r + r
    minv = jnp.full((TB,), jnp.inf, dtype=jnp.float32)
    mini = jnp.zeros((TB,), dtype=jnp.int32)
    io = lax.broadcasted_iota(jnp.int32, (TB, KT), 1)
    for k in range(NKT):
        cbt = cb_ref[pl.ds(k * KT, KT), :]
        c2 = c2_ref[0, pl.ds(k * KT, KT)]
        cross2 = lax.dot_general(rd, cbt, (((1,), (1,)), ((), ())),
                                 preferred_element_type=jnp.float32)
        d2 = (r2[:, None] + c2[None, :]) - cross2
        tmin = _fold_min(d2)
        targ = _fold_min(jnp.where(d2 == tmin[:, None], io, K)) + k * KT
        upd = tmin < minv
        minv = jnp.where(upd, tmin, minv)
        mini = jnp.where(upd, targ, mini)
    idx_ref[0, 0, :] = mini


def _c2_body(cb_ref, c2_ref):
    for k in range(NKT):
        cbt = cb_ref[pl.ds(k * KT, KT), :]
        c2_ref[0, pl.ds(k * KT, KT)] = jnp.sum(cbt * cbt, axis=1)


def _c2(cb):
    return pl.pallas_call(
        _c2_body,
        grid=(1,),
        in_specs=[pl.BlockSpec((K, D), lambda i: (0, 0))],
        out_specs=pl.BlockSpec((1, K), lambda i: (0, 0)),
        out_shape=jax.ShapeDtypeStruct((1, K), jnp.float32),
    )(cb)


def _vq_first_body(r_ref, cb_ref, c2_ref, idx_ref):
    _argmin_into(r_ref[...], cb_ref, c2_ref, idx_ref)


def _vq_next_body(r_ref, rows_ref, cb_ref, c2_ref, idx_ref, rout_ref):
    r = r_ref[...] - rows_ref[...]
    rout_ref[...] = r
    _argmin_into(r, cb_ref, c2_ref, idx_ref)


def _vq_last_body(r_ref, rows_ref, x_ref, cb_ref, c2_ref, idx_ref, rec01_ref):
    r = r_ref[...] - rows_ref[...]
    rec01_ref[...] = x_ref[...] - r
    _argmin_into(r, cb_ref, c2_ref, idx_ref)


def _vq_first(r, cb, c2):
    return pl.pallas_call(
        _vq_first_body,
        grid=(NB,),
        in_specs=[
            pl.BlockSpec((TB, D), lambda i: (i, 0)),
            pl.BlockSpec((K, D), lambda i: (0, 0)),
            pl.BlockSpec((1, K), lambda i: (0, 0)),
        ],
        out_specs=pl.BlockSpec((1, 1, TB), lambda i: (i, 0, 0)),
        out_shape=jax.ShapeDtypeStruct((NB, 1, TB), jnp.int32),
    )(r, cb, c2)


def _vq_next(r_prev, rows_prev, cb, c2):
    return pl.pallas_call(
        _vq_next_body,
        grid=(NB,),
        in_specs=[
            pl.BlockSpec((TB, D), lambda i: (i, 0)),
            pl.BlockSpec((TB, D), lambda i: (i, 0)),
            pl.BlockSpec((K, D), lambda i: (0, 0)),
            pl.BlockSpec((1, K), lambda i: (0, 0)),
        ],
        out_specs=[
            pl.BlockSpec((1, 1, TB), lambda i: (i, 0, 0)),
            pl.BlockSpec((TB, D), lambda i: (i, 0)),
        ],
        out_shape=[
            jax.ShapeDtypeStruct((NB, 1, TB), jnp.int32),
            jax.ShapeDtypeStruct((N, D), jnp.float32),
        ],
    )(r_prev, rows_prev, cb, c2)


def _vq_last(r_prev, rows_prev, x, cb, c2):
    return pl.pallas_call(
        _vq_last_body,
        grid=(NB,),
        in_specs=[
            pl.BlockSpec((TB, D), lambda i: (i, 0)),
            pl.BlockSpec((TB, D), lambda i: (i, 0)),
            pl.BlockSpec((TB, D), lambda i: (i, 0)),
            pl.BlockSpec((K, D), lambda i: (0, 0)),
            pl.BlockSpec((1, K), lambda i: (0, 0)),
        ],
        out_specs=[
            pl.BlockSpec((1, 1, TB), lambda i: (i, 0, 0)),
            pl.BlockSpec((TB, D), lambda i: (i, 0)),
        ],
        out_shape=[
            jax.ShapeDtypeStruct((NB, 1, TB), jnp.int32),
            jax.ShapeDtypeStruct((N, D), jnp.float32),
        ],
    )(r_prev, rows_prev, x, cb, c2)


def _sc_gather_body(cb_hbm, idx_hbm, out_hbm, idx_v, rows_v, sem):
    wid = lax.axis_index("s") * NC + lax.axis_index("c")
    pltpu.sync_copy(idx_hbm.at[wid], idx_v)
    copies = [
        pltpu.async_copy(cb_hbm.at[idx_v.at[j]], rows_v.at[j], sem)
        for j in range(NCH)
    ]
    for c in copies:
        c.wait()
    pltpu.sync_copy(rows_v, out_hbm.at[wid])


def _sc_gather_add_body(cb_hbm, idx_hbm, base_hbm, out_hbm, idx_v, rows_v,
                        base_v, sem):
    wid = lax.axis_index("s") * NC + lax.axis_index("c")
    pltpu.sync_copy(idx_hbm.at[wid], idx_v)
    copies = [
        pltpu.async_copy(cb_hbm.at[idx_v.at[j]], rows_v.at[j], sem)
        for j in range(NCH)
    ]
    pltpu.sync_copy(base_hbm.at[wid], base_v)
    for c in copies:
        c.wait()

    def _add_row(i, _):
        for j in range(NCH):
            for l in range(D // 16):
                s = pl.ds(l * 16, 16)
                rows_v[j, i, s] = rows_v[j, i, s] + base_v[j, i, s]
        return 0

    lax.fori_loop(0, CH, _add_row, 0)
    pltpu.sync_copy(rows_v, out_hbm.at[wid])


@functools.cache
def _sc_gather_kernel():
    return pl.kernel(
        _sc_gather_body,
        out_type=jax.ShapeDtypeStruct((NW, NCH, CH, D), jnp.float32),
        mesh=plsc.VectorSubcoreMesh(core_axis_name="c", subcore_axis_name="s"),
        scratch_types=[
            pltpu.VMEM((NCH, CH), jnp.int32),
            pltpu.VMEM((NCH, CH, D), jnp.float32),
            pltpu.SemaphoreType.DMA,
        ],
    )


@functools.cache
def _sc_gather_add_kernel():
    return pl.kernel(
        _sc_gather_add_body,
        out_type=jax.ShapeDtypeStruct((NW, NCH, CH, D), jnp.float32),
        mesh=plsc.VectorSubcoreMesh(core_axis_name="c", subcore_axis_name="s"),
        scratch_types=[
            pltpu.VMEM((NCH, CH), jnp.int32),
            pltpu.VMEM((NCH, CH, D), jnp.float32),
            pltpu.VMEM((NCH, CH, D), jnp.float32),
            pltpu.SemaphoreType.DMA,
        ],
    )


def _gather(cb, idx_flat):
    out = _sc_gather_kernel()(cb, idx_flat.reshape(NW, NCH, CH))
    return out.reshape(N, D)


def _gather_add(cb, idx_flat, base):
    out = _sc_gather_add_kernel()(
        cb, idx_flat.reshape(NW, NCH, CH), base.reshape(NW, NCH, CH, D))
    return out.reshape(N, D)


def kernel(multimodal_features, codebook_0, codebook_1, codebook_2):
    x = multimodal_features.reshape(N, D)

    c20 = _c2(codebook_0)
    idx0 = _vq_first(x, codebook_0, c20).reshape(N)
    c21 = _c2(codebook_1)
    rows0 = _gather(codebook_0, idx0)

    idx1, r1 = _vq_next(x, rows0, codebook_1, c21)
    idx1 = idx1.reshape(N)
    c22 = _c2(codebook_2)
    rows1 = _gather(codebook_1, idx1)

    idx2, rec01 = _vq_last(r1, rows1, x, codebook_2, c22)
    idx2 = idx2.reshape(N)
    recon = _gather_add(codebook_2, idx2, rec01).reshape(B, T, D)
    semantic_ids = jnp.stack([idx0, idx1, idx2], axis=-1).reshape(B, T, 3)
    return semantic_ids, recon


# speedup vs baseline: 1.0602x; 1.0339x over previous
import functools

import jax
import jax.numpy as jnp
from jax import lax
from jax.experimental import pallas as pl
from jax.experimental.pallas import tpu as pltpu
from jax.experimental.pallas import tpu_sc as plsc

B, T, D, K = 8, 576, 256, 8192
N = B * T
TB = 512
NB = N // TB
KT = 2048
NKT = K // KT

NC = 2
NS = 16
NW = NC * NS
BPW = N // NW
NCH = 2
CH = BPW // NCH


def _fold_min(x):
    n = x.shape[1]
    while n > 128:
        h = n // 2
        x = jnp.minimum(x[:, :h], x[:, h:])
        n = h
    return jnp.min(x, axis=1)


def _argmin_into(r, cb_ref, c2_ref, idx_ref):
    r2 = jnp.sum(r * r, axis=1)
    rd = r + r
    minv = jnp.full((TB,), jnp.inf, dtype=jnp.float32)
    mini = jnp.zeros((TB,), dtype=jnp.int32)
    io = lax.broadcasted_iota(jnp.int32, (TB, KT), 1)
    for k in range(NKT):
        cbt = cb_ref[pl.ds(k * KT, KT), :]
        c2 = c2_ref[0, pl.ds(k * KT, KT)]
        cross2 = lax.dot_general(rd, cbt, (((1,), (1,)), ((), ())),
                                 preferred_element_type=jnp.float32)
        d2 = (r2[:, None] + c2[None, :]) - cross2
        tmin = _fold_min(d2)
        targ = _fold_min(jnp.where(d2 == tmin[:, None], io, K)) + k * KT
        upd = tmin < minv
        minv = jnp.where(upd, tmin, minv)
        mini = jnp.where(upd, targ, mini)
    idx_ref[...] = mini


def _c2_body(cb_ref, c2_ref):
    for k in range(NKT):
        cbt = cb_ref[pl.ds(k * KT, KT), :]
        c2_ref[0, pl.ds(k * KT, KT)] = jnp.sum(cbt * cbt, axis=1)


def _c2(cb):
    return pl.pallas_call(
        _c2_body,
        grid=(1,),
        in_specs=[pl.BlockSpec((K, D), lambda i: (0, 0))],
        out_specs=pl.BlockSpec((1, K), lambda i: (0, 0)),
        out_shape=jax.ShapeDtypeStruct((1, K), jnp.float32),
    )(cb)


def _vq_first_body(r_ref, cb_ref, idx_ref, c2_ref):
    @pl.when(pl.program_id(0) == 0)
    def _():
        _c2_body(cb_ref, c2_ref)

    _argmin_into(r_ref[...], cb_ref, c2_ref, idx_ref)


def _vq_next_body(r_ref, rows_ref, cb_ref, c2_ref, idx_ref, rout_ref):
    r = r_ref[...] - rows_ref[...]
    rout_ref[...] = r
    _argmin_into(r, cb_ref, c2_ref, idx_ref)


def _vq_last_body(r_ref, rows_ref, x_ref, cb_ref, c2_ref, idx_ref, rec01_ref):
    r = r_ref[...] - rows_ref[...]
    rec01_ref[...] = x_ref[...] - r
    _argmin_into(r, cb_ref, c2_ref, idx_ref)


def _vq_first(r, cb):
    return pl.pallas_call(
        _vq_first_body,
        grid=(NB,),
        in_specs=[
            pl.BlockSpec((TB, D), lambda i: (i, 0)),
            pl.BlockSpec((K, D), lambda i: (0, 0)),
        ],
        out_specs=pl.BlockSpec((TB,), lambda i: (i,)),
        out_shape=jax.ShapeDtypeStruct((N,), jnp.int32),
        scratch_shapes=[pltpu.VMEM((1, K), jnp.float32)],
    )(r, cb)


def _vq_next(r_prev, rows_prev, cb, c2):
    return pl.pallas_call(
        _vq_next_body,
        grid=(NB,),
        in_specs=[
            pl.BlockSpec((TB, D), lambda i: (i, 0)),
            pl.BlockSpec((TB, D), lambda i: (i, 0)),
            pl.BlockSpec((K, D), lambda i: (0, 0)),
            pl.BlockSpec((1, K), lambda i: (0, 0)),
        ],
        out_specs=[
            pl.BlockSpec((TB,), lambda i: (i,)),
            pl.BlockSpec((TB, D), lambda i: (i, 0)),
        ],
        out_shape=[
            jax.ShapeDtypeStruct((N,), jnp.int32),
            jax.ShapeDtypeStruct((N, D), jnp.float32),
        ],
    )(r_prev, rows_prev, cb, c2)


def _vq_last(r_prev, rows_prev, x, cb, c2):
    return pl.pallas_call(
        _vq_last_body,
        grid=(NB,),
        in_specs=[
            pl.BlockSpec((TB, D), lambda i: (i, 0)),
            pl.BlockSpec((TB, D), lambda i: (i, 0)),
            pl.BlockSpec((TB, D), lambda i: (i, 0)),
            pl.BlockSpec((K, D), lambda i: (0, 0)),
            pl.BlockSpec((1, K), lambda i: (0, 0)),
        ],
        out_specs=[
            pl.BlockSpec((TB,), lambda i: (i,)),
            pl.BlockSpec((TB, D), lambda i: (i, 0)),
        ],
        out_shape=[
            jax.ShapeDtypeStruct((N,), jnp.int32),
            jax.ShapeDtypeStruct((N, D), jnp.float32),
        ],
    )(r_prev, rows_prev, x, cb, c2)


def _sc_gather_body(cb_hbm, idx_hbm, out_hbm, idx_v, rows_v, sem):
    wid = lax.axis_index("s") * NC + lax.axis_index("c")
    for j in range(NCH):
        pltpu.sync_copy(idx_hbm.at[pl.ds(wid * BPW + j * CH, CH)], idx_v.at[j])
    copies = [
        pltpu.async_copy(cb_hbm.at[idx_v.at[j]], rows_v.at[j], sem)
        for j in range(NCH)
    ]
    for c in copies:
        c.wait()
    pltpu.sync_copy(rows_v, out_hbm.at[wid])


def _sc_gather_add_body(cb_hbm, idx_hbm, base_hbm, out_hbm, idx_v, rows_v,
                        base_v, sem):
    wid = lax.axis_index("s") * NC + lax.axis_index("c")
    for j in range(NCH):
        pltpu.sync_copy(idx_hbm.at[pl.ds(wid * BPW + j * CH, CH)], idx_v.at[j])
    copies = [
        pltpu.async_copy(cb_hbm.at[idx_v.at[j]], rows_v.at[j], sem)
        for j in range(NCH)
    ]
    pltpu.sync_copy(base_hbm.at[wid], base_v)
    for c in copies:
        c.wait()

    def _add_row(i, _):
        for j in range(NCH):
            for l in range(D // 16):
                s = pl.ds(l * 16, 16)
                rows_v[j, i, s] = rows_v[j, i, s] + base_v[j, i, s]
        return 0

    lax.fori_loop(0, CH, _add_row, 0)
    pltpu.sync_copy(rows_v, out_hbm.at[wid])


@functools.cache
def _sc_gather_kernel():
    return pl.kernel(
        _sc_gather_body,
        out_type=jax.ShapeDtypeStruct((NW, NCH, CH, D), jnp.float32),
        mesh=plsc.VectorSubcoreMesh(core_axis_name="c", subcore_axis_name="s"),
        scratch_types=[
            pltpu.VMEM((NCH, CH), jnp.int32),
            pltpu.VMEM((NCH, CH, D), jnp.float32),
            pltpu.SemaphoreType.DMA,
        ],
    )


@functools.cache
def _sc_gather_add_kernel():
    return pl.kernel(
        _sc_gather_add_body,
        out_type=jax.ShapeDtypeStruct((NW, NCH, CH, D), jnp.float32),
        mesh=plsc.VectorSubcoreMesh(core_axis_name="c", subcore_axis_name="s"),
        scratch_types=[
            pltpu.VMEM((NCH, CH), jnp.int32),
            pltpu.VMEM((NCH, CH, D), jnp.float32),
            pltpu.VMEM((NCH, CH, D), jnp.float32),
            pltpu.SemaphoreType.DMA,
        ],
    )


def _gather(cb, idx_flat):
    out = _sc_gather_kernel()(cb, idx_flat)
    return out.reshape(N, D)


def _gather_add(cb, idx_flat, base):
    out = _sc_gather_add_kernel()(
        cb, idx_flat, base.reshape(NW, NCH, CH, D))
    return out.reshape(N, D)


def kernel(multimodal_features, codebook_0, codebook_1, codebook_2):
    x = multimodal_features.reshape(N, D)

    idx0 = _vq_first(x, codebook_0)
    c21 = _c2(codebook_1)
    rows0 = _gather(codebook_0, idx0)

    idx1, r1 = _vq_next(x, rows0, codebook_1, c21)
    c22 = _c2(codebook_2)
    rows1 = _gather(codebook_1, idx1)

    idx2, rec01 = _vq_last(r1, rows1, x, codebook_2, c22)
    recon = _gather_add(codebook_2, idx2, rec01).reshape(B, T, D)
    semantic_ids = jnp.stack([idx0, idx1, idx2], axis=-1).reshape(B, T, 3)
    return semantic_ids, recon
